# Initial kernel scaffold; baseline (speedup 1.0000x reference)
import functools

import jax
import jax.numpy as jnp
from jax import lax
from jax.experimental import pallas as pl
from jax.experimental.pallas import tpu as pltpu
from jax.experimental.pallas import tpu_sc as plsc

VOCAB = 1000
N_POS = 1024 * 50
NC, NS, L = 2, 16, 16
NW = NC * NS
PER_W = N_POS // NW
CHUNK = 32
N_CHUNKS = PER_W // CHUNK


def _lse_body(table_ref, lse_ref):
    t = table_ref[...]
    m = jnp.max(t, axis=1)
    s = jnp.sum(jnp.exp(t - m[:, None]), axis=1)
    lse_ref[...] = m + jnp.log(s)


def _finish_body(parts_ref, loss_ref):
    loss_ref[0, 0] = jnp.sum(parts_ref[...]) * (1.0 / N_POS)


def _sc_body(table_hbm, idx_hbm, tgt_hbm, lse_hbm,
             out_hbm, part_hbm,
             idx_v, tgt_v, lse_v, rows_v, acc_v, sem):
    wid = lax.axis_index("s") * NC + lax.axis_index("c")
    base = wid * PER_W
    pltpu.sync_copy(idx_hbm.at[pl.ds(base, PER_W)], idx_v)
    pltpu.sync_copy(tgt_hbm.at[pl.ds(base, PER_W)], tgt_v)
    pltpu.sync_copy(lse_hbm, lse_v)

    lane = lax.iota(jnp.int32, L)

    def chunk_body(g, acc):
        off = g * CHUNK
        idx_sl = idx_v.at[pl.ds(off, CHUNK)]
        pltpu.async_copy(table_hbm.at[idx_sl], rows_v, sem).wait()
        pltpu.sync_copy(rows_v, out_hbm.at[pl.ds(base + off, CHUNK)])
        for j in range(CHUNK // L):
            rid = lane + (j * L)
            cols = tgt_v[pl.ds(off + j * L, L)]
            ins = idx_v[pl.ds(off + j * L, L)]
            tv = plsc.load_gather(rows_v, [rid, cols])
            ls = plsc.load_gather(lse_v, [ins])
            acc = acc + (ls - tv)
        return acc

    acc = lax.fori_loop(0, N_CHUNKS, chunk_body, jnp.zeros((L,), jnp.float32))
    acc_v[...] = acc
    pltpu.sync_copy(acc_v, part_hbm.at[wid])


def kernel(inputs, targets, table):
    B, Ln = inputs.shape
    idx_flat = inputs.reshape(-1)
    tgt_flat = targets.reshape(-1)

    lse = pl.pallas_call(
        _lse_body,
        out_shape=jax.ShapeDtypeStruct((VOCAB,), jnp.float32),
    )(table)

    mesh = plsc.VectorSubcoreMesh(core_axis_name="c", subcore_axis_name="s")
    sc = pl.kernel(
        _sc_body,
        out_type=(
            jax.ShapeDtypeStruct((N_POS, VOCAB), jnp.float32),
            jax.ShapeDtypeStruct((NW, L), jnp.float32),
        ),
        mesh=mesh,
        scratch_types=[
            pltpu.VMEM((PER_W,), jnp.int32),
            pltpu.VMEM((PER_W,), jnp.int32),
            pltpu.VMEM((VOCAB,), jnp.float32),
            pltpu.VMEM((CHUNK, VOCAB), jnp.float32),
            pltpu.VMEM((L,), jnp.float32),
            pltpu.SemaphoreType.DMA,
        ],
    )
    logits_flat, parts = sc(table, idx_flat, tgt_flat, lse)

    loss = pl.pallas_call(
        _finish_body,
        out_shape=jax.ShapeDtypeStruct((1, 1), jnp.float32),
    )(parts)[0, 0]

    return logits_flat.reshape(B, Ln, VOCAB), loss


# speedup vs baseline: 1.4823x; 1.4823x over previous
import jax
import jax.numpy as jnp
from jax import lax
from jax.experimental import pallas as pl
from jax.experimental.pallas import tpu as pltpu
from jax.experimental.pallas import tpu_sc as plsc

VOCAB = 1000
N_POS = 1024 * 50
NC, NS, L = 2, 16, 16
NW = NC * NS
PER_W = N_POS // NW
CHUNK = 64
N_CHUNKS = PER_W // CHUNK


def _lse_body(table_ref, lse_ref, tcopy_ref):
    t = table_ref[...]
    m = jnp.max(t, axis=1)
    s = jnp.sum(jnp.exp(t - m[:, None]), axis=1)
    lse_ref[...] = m + jnp.log(s)
    tcopy_ref[...] = t


def _finish_body(parts_ref, loss_ref):
    loss_ref[...] = jnp.sum(parts_ref[...], keepdims=True) * (1.0 / N_POS)


def _sc_body(table_hbm, tflat_hbm, idx_hbm, tgt_hbm, lse_hbm,
             out_hbm, part_hbm,
             idx_v, fidx_v, rows_v, tv_v, lse_g_v, acc_v, sem, sem2, sem3):
    wid = lax.axis_index("s") * NC + lax.axis_index("c")
    base = wid * PER_W
    pltpu.sync_copy(idx_hbm.at[pl.ds(base, PER_W)], idx_v)
    pltpu.sync_copy(tgt_hbm.at[pl.ds(base, PER_W)], fidx_v)

    def fbody(i, _):
        sl = pl.ds(i * L, L)
        fidx_v[sl] = idx_v[sl] * VOCAB + fidx_v[sl]
        return 0

    lax.fori_loop(0, PER_W // L, fbody, 0)

    def chunk_body(g, acc):
        off = g * CHUNK
        idx_sl = idx_v.at[pl.ds(off, CHUNK)]
        cp_rows = pltpu.async_copy(table_hbm.at[idx_sl], rows_v, sem)
        cp_tv = pltpu.async_copy(
            tflat_hbm.at[fidx_v.at[pl.ds(off, CHUNK)]], tv_v, sem2)
        cp_lse = pltpu.async_copy(lse_hbm.at[idx_sl], lse_g_v, sem3)
        cp_rows.wait()
        pltpu.sync_copy(rows_v, out_hbm.at[pl.ds(base + off, CHUNK)])
        cp_tv.wait()
        cp_lse.wait()
        for j in range(CHUNK // L):
            sl = pl.ds(j * L, L)
            acc = acc + (lse_g_v[sl] - tv_v[sl])
        return acc

    acc = lax.fori_loop(0, N_CHUNKS, chunk_body, jnp.zeros((L,), jnp.float32))
    acc_v[...] = acc
    pltpu.sync_copy(acc_v, part_hbm.at[wid])


def kernel(inputs, targets, table):
    B, Ln = inputs.shape
    idx_flat = inputs.reshape(-1)
    tgt_flat = targets.reshape(-1)

    lse, tcopy = pl.pallas_call(
        _lse_body,
        out_shape=(
            jax.ShapeDtypeStruct((VOCAB,), jnp.float32),
            jax.ShapeDtypeStruct((VOCAB, VOCAB), jnp.float32),
        ),
    )(table)
    table_flat = tcopy.reshape(-1)

    mesh = plsc.VectorSubcoreMesh(core_axis_name="c", subcore_axis_name="s")
    sc = pl.kernel(
        _sc_body,
        out_type=(
            jax.ShapeDtypeStruct((N_POS, VOCAB), jnp.float32),
            jax.ShapeDtypeStruct((NW, L), jnp.float32),
        ),
        mesh=mesh,
        compiler_params=pltpu.CompilerParams(use_tc_tiling_on_sc=False),
        scratch_types=[
            pltpu.VMEM((PER_W,), jnp.int32),
            pltpu.VMEM((PER_W,), jnp.int32),
            pltpu.VMEM((CHUNK, VOCAB), jnp.float32),
            pltpu.VMEM((CHUNK,), jnp.float32),
            pltpu.VMEM((CHUNK,), jnp.float32),
            pltpu.VMEM((L,), jnp.float32),
            pltpu.SemaphoreType.DMA,
            pltpu.SemaphoreType.DMA,
            pltpu.SemaphoreType.DMA,
        ],
    )
    logits_flat, parts = sc(table, table_flat, idx_flat, tgt_flat, lse)

    loss = pl.pallas_call(
        _finish_body,
        out_shape=jax.ShapeDtypeStruct((1, 1), jnp.float32),
    )(parts)[0, 0]

    return logits_flat.reshape(B, Ln, VOCAB), loss
